# Initial kernel scaffold; baseline (speedup 1.0000x reference)
import jax
import jax.numpy as jnp
from jax.experimental import pallas as pl


def kernel(xyz_dis, scaling, opacity, neural_features, lap_indices):
    raise NotImplementedError("write your pallas kernel here")


# speedup vs baseline: 29.7990x; 29.7990x over previous
import functools

import jax
import jax.numpy as jnp
import numpy as np
from jax import lax
from jax.experimental import pallas as pl
from jax.experimental.pallas import tpu as pltpu
from jax.experimental.pallas import tpu_sc as plsc

_N = 100000
_K = 9
_D = 48
_L = 16
_NC = 2
_NS = 16
_NW = _NC * _NS
_CP = 14
_IPT = _CP * _K
_NBUF = 4

_NPAD = ((_N + _NW * _CP - 1) // (_NW * _CP)) * (_NW * _CP)
_TPW = _NPAD // (_NW * _CP)
_NTR = _NW * _TPW


def _body(tbl, idxh, wch, out, idx_v, rows_v, wv_v, out_v, s0, s1, s2, s3):
    sems = (s0, s1, s2, s3)
    wid = lax.axis_index("s") * _NC + lax.axis_index("c")

    pltpu.sync_copy(idxh.at[pl.ds(wid * _TPW, _TPW)], idx_v)
    pltpu.sync_copy(wch, wv_v)
    wvec = [wv_v[pl.ds(g * _L, _L)] for g in range(3)]

    for b in range(_NBUF):
        pltpu.async_copy(tbl.at[idx_v.at[b]], rows_v.at[b], sems[b])

    def outer(t2, acc):
        t = t2 * _NBUF
        for b in range(_NBUF):
            pltpu.make_async_copy(tbl.at[idx_v.at[b]], rows_v.at[b],
                                  sems[b]).wait()

            def point(p, a, b=b):
                r = p * _K
                for g in range(3):
                    col = pl.ds(g * _L, _L)
                    c = rows_v[b, r, col]
                    s = rows_v[b, r + 1, col]
                    for j in range(2, _K):
                        s = s + rows_v[b, r + j, col]
                    d = c - 0.125 * s
                    a = a + (d * d) * wvec[g]
                return a

            acc = lax.fori_loop(0, _CP, point, acc)

            tn = t + b + _NBUF

            @pl.when(tn < _TPW)
            def _(b=b, tn=tn):
                pltpu.async_copy(tbl.at[idx_v.at[tn]], rows_v.at[b], sems[b])

        return acc

    acc = lax.fori_loop(0, _TPW // _NBUF, outer,
                        jnp.zeros((_L,), jnp.float32))
    out_v[...] = acc
    pltpu.sync_copy(out_v, out.at[pl.ds(wid * _L, _L)])


_kern = functools.partial(
    pl.kernel,
    out_type=jax.ShapeDtypeStruct((_NW * _L,), jnp.float32),
    mesh=plsc.VectorSubcoreMesh(core_axis_name="c", subcore_axis_name="s"),
    scratch_types=[
        pltpu.VMEM((_TPW, _IPT), jnp.int32),
        pltpu.VMEM((_NBUF, _IPT, _D), jnp.float32),
        pltpu.VMEM((_D,), jnp.float32),
        pltpu.VMEM((_L,), jnp.float32),
        pltpu.SemaphoreType.DMA,
        pltpu.SemaphoreType.DMA,
        pltpu.SemaphoreType.DMA,
        pltpu.SemaphoreType.DMA,
    ],
    compiler_params=pltpu.CompilerParams(use_tc_tiling_on_sc=False),
)(_body)


_WCOL = np.zeros((_D,), np.float32)
_WCOL[0:3] = 1.0 / (_N * 3)
_WCOL[3:6] = 1.0 / (_N * 3)
_WCOL[6] = 1.0 / _N
_WCOL[7:39] = 1.0 / (_N * 32)


def kernel(xyz_dis, scaling, opacity, neural_features, lap_indices):
    n = xyz_dis.shape[0]
    table = jnp.concatenate(
        [xyz_dis, scaling, opacity, neural_features,
         jnp.zeros((n, _D - 39), jnp.float32)], axis=1)
    idx = lap_indices.astype(jnp.int32)
    idx = jnp.concatenate(
        [idx, jnp.zeros((_NPAD - n, _K), jnp.int32)], axis=0)
    idx2 = idx.reshape(_NTR, _IPT)
    parts = _kern(table, idx2, jnp.asarray(_WCOL))
    return jnp.sum(parts)


# speedup vs baseline: 39.0301x; 1.3098x over previous
import functools

import jax
import jax.numpy as jnp
from jax import lax
from jax.experimental import pallas as pl
from jax.experimental.pallas import tpu as pltpu
from jax.experimental.pallas import tpu_sc as plsc

_N = 100000
_K = 9
_DF = 32
_DS = 16
_L = 16
_NC = 2
_NS = 16
_NW = _NC * _NS
_CP = 14
_IPT = _CP * _K
_NBUF = 4

_NPAD = ((_N + _NW * _CP - 1) // (_NW * _CP)) * (_NW * _CP)
_TPW = _NPAD // (_NW * _CP)
_NTR = _NW * _TPW

_WF = 1.0 / (_N * _DF)
_W3 = 1.0 / (_N * 3)
_W1 = 1.0 / _N


def _body(feat, small, idxh, out, idx_v, rf_v, rs_v, out_v, s0, s1, s2, s3):
    sems = (s0, s1, s2, s3)
    wid = lax.axis_index("s") * _NC + lax.axis_index("c")

    pltpu.sync_copy(idxh.at[pl.ds(wid * _TPW, _TPW)], idx_v)

    io = lax.iota(jnp.int32, _L)
    wsm = jnp.where(io < 6, jnp.float32(_W3),
                    jnp.where(io == 6, jnp.float32(_W1),
                              jnp.zeros((_L,), jnp.float32)))

    def fire(t, b):
        pltpu.async_copy(feat.at[idx_v.at[t]], rf_v.at[b], sems[b])
        pltpu.async_copy(small.at[idx_v.at[t]], rs_v.at[b], sems[b])

    for b in range(_NBUF):
        fire(b, b)

    def outer(t2, acc):
        t = t2 * _NBUF
        for b in range(_NBUF):
            pltpu.make_async_copy(feat.at[idx_v.at[b]], rf_v.at[b],
                                  sems[b]).wait()
            pltpu.make_async_copy(small.at[idx_v.at[b]], rs_v.at[b],
                                  sems[b]).wait()

            def point(p, a, b=b):
                r = p * _K
                for g in range(2):
                    col = pl.ds(g * _L, _L)
                    c = rf_v[b, r, col]
                    s = rf_v[b, r + 1, col]
                    for j in range(2, _K):
                        s = s + rf_v[b, r + j, col]
                    d = c - 0.125 * s
                    a = a + (d * d) * _WF
                c = rs_v[b, r, :]
                s = rs_v[b, r + 1, :]
                for j in range(2, _K):
                    s = s + rs_v[b, r + j, :]
                d = c - 0.125 * s
                return a + (d * d) * wsm

            acc = lax.fori_loop(0, _CP, point, acc)

            tn = t + b + _NBUF

            @pl.when(tn < _TPW)
            def _(b=b, tn=tn):
                fire(tn, b)

        return acc

    acc = lax.fori_loop(0, _TPW // _NBUF, outer,
                        jnp.zeros((_L,), jnp.float32))
    out_v[...] = acc
    pltpu.sync_copy(out_v, out.at[pl.ds(wid * _L, _L)])


_kern = functools.partial(
    pl.kernel,
    out_type=jax.ShapeDtypeStruct((_NW * _L,), jnp.float32),
    mesh=plsc.VectorSubcoreMesh(core_axis_name="c", subcore_axis_name="s"),
    scratch_types=[
        pltpu.VMEM((_TPW, _IPT), jnp.int32),
        pltpu.VMEM((_NBUF, _IPT, _DF), jnp.float32),
        pltpu.VMEM((_NBUF, _IPT, _DS), jnp.float32),
        pltpu.VMEM((_L,), jnp.float32),
        pltpu.SemaphoreType.DMA,
        pltpu.SemaphoreType.DMA,
        pltpu.SemaphoreType.DMA,
        pltpu.SemaphoreType.DMA,
    ],
    compiler_params=pltpu.CompilerParams(use_tc_tiling_on_sc=False),
)(_body)


def kernel(xyz_dis, scaling, opacity, neural_features, lap_indices):
    n = xyz_dis.shape[0]
    small = jnp.concatenate(
        [xyz_dis, scaling, opacity, jnp.zeros((n, _DS - 7), jnp.float32)],
        axis=1)
    idx = lap_indices.astype(jnp.int32)
    idx = jnp.concatenate(
        [idx, jnp.zeros((_NPAD - n, _K), jnp.int32)], axis=0)
    idx2 = idx.reshape(_NTR, _IPT)
    parts = _kern(neural_features, small, idx2)
    return jnp.sum(parts)
